# baseline (speedup 1.0000x reference)
import functools

import jax
import jax.numpy as jnp
from jax import lax
from jax.experimental import pallas as pl
from jax.experimental.pallas import tpu as pltpu
from jax.experimental.pallas import tpu_sc as plsc

N = 10000
E = 320000
F = 128
K = 3
L = 3

NC = 2
NS = 16
NW = NC * NS
EPW = E // NW
C = 80
NCHUNK = EPW // C

RPT = 640
QN = RPT // C
QN_LAST = (N - 15 * RPT) // C

_MESH = plsc.VectorSubcoreMesh(
    core_axis_name="c", subcore_axis_name="s", num_cores=NC, num_subcores=NS)

_GD = lax.GatherDimensionNumbers(
    offset_dims=(), collapsed_slice_dims=(0,), start_index_map=(0,))


def _vsplat(vec, lane):
    idx = jnp.full((16,), lane, jnp.int32)
    return lax.gather(vec, idx[:, None], _GD, (1,),
                      mode=lax.GatherScatterMode.PROMISE_IN_BOUNDS)


def _staged(s, fn):
    @pl.when(s < 15)
    def _():
        lax.fori_loop(0, QN, lambda q, _: (fn(q),)[1:], ())

    @pl.when(s == 15)
    def _():
        lax.fori_loop(0, QN_LAST, lambda q, _: (fn(q),)[1:], ())


@functools.partial(
    pl.kernel,
    out_type=[jax.ShapeDtypeStruct((N,), jnp.float32),
              jax.ShapeDtypeStruct((N,), jnp.float32)],
    mesh=_MESH,
    scratch_types=[
        pltpu.VMEM((NCHUNK, C), jnp.int32),
        pltpu.VMEM((EPW,), jnp.float32),
        pltpu.VMEM((C,), jnp.float32),
        pltpu.SemaphoreType.DMA,
        pltpu.VMEM_SHARED((N,), jnp.float32),
    ],
)
def _deg(col3_hbm, ew_hbm, out0_hbm, out1_hbm, col_v, w_v, zb_v, sem, acc):
    c = lax.axis_index("c")
    s = lax.axis_index("s")
    wid = s * NC + c
    pltpu.sync_copy(col3_hbm.at[wid], col_v)
    pltpu.sync_copy(ew_hbm.at[pl.ds(wid * EPW, EPW)], w_v)
    for j in range(C // 16):
        zb_v[pl.ds(j * 16, 16)] = jnp.zeros((16,), jnp.float32)
    _staged(s, lambda q: pltpu.sync_copy(zb_v, acc.at[pl.ds(s * RPT + q * C, C)]))
    plsc.subcore_barrier()

    def fire(i, _):
        pltpu.async_copy(w_v.at[pl.ds(i * C, C)], acc.at[col_v.at[i]], sem,
                         add=True)
        return ()

    def drain(i, _):
        pltpu.make_async_copy(
            w_v.at[pl.ds(i * C, C)], acc.at[col_v.at[i]], sem).wait()
        return ()

    lax.fori_loop(0, NCHUNK, fire, ())
    lax.fori_loop(0, NCHUNK, drain, ())
    plsc.subcore_barrier()

    def rd(q):
        off = s * RPT + q * C
        pltpu.sync_copy(acc.at[pl.ds(off, C)], zb_v)

        @pl.when(c == 0)
        def _():
            pltpu.sync_copy(zb_v, out0_hbm.at[pl.ds(off, C)])

        @pl.when(c == 1)
        def _():
            pltpu.sync_copy(zb_v, out1_hbm.at[pl.ds(off, C)])

    _staged(s, rd)


@functools.partial(
    pl.kernel,
    out_type=jax.ShapeDtypeStruct((NC, N, F), jnp.float32),
    mesh=_MESH,
    scratch_types=[
        pltpu.VMEM((EPW,), jnp.int32),
        pltpu.VMEM((3, C), jnp.int32),
        pltpu.VMEM((3, C), jnp.float32),
        pltpu.VMEM((C, F), jnp.float32),
        pltpu.VMEM((C, F), jnp.float32),
        pltpu.VMEM((C, F), jnp.float32),
        pltpu.SemaphoreType.DMA,
        pltpu.SemaphoreType.DMA,
        pltpu.SemaphoreType.DMA,
        pltpu.SemaphoreType.DMA,
        pltpu.SemaphoreType.DMA,
        pltpu.SemaphoreType.DMA,
        pltpu.SemaphoreType.DMA,
        pltpu.SemaphoreType.DMA,
        pltpu.SemaphoreType.DMA,
        pltpu.VMEM_SHARED((N, F), jnp.float32),
    ],
)
def _prop(g_hbm, row_hbm, col_hbm, ew_hbm, out_hbm,
          row_v, col_v, w_v, bufa, bufb, bufc,
          sga, sgb, sgc, sma, smb, smc, ssa, ssb, ssc, acc):
    c = lax.axis_index("c")
    s = lax.axis_index("s")
    wid = s * NC + c
    base = wid * EPW
    BUF = [bufa, bufb, bufc]
    SG = [sga, sgb, sgc]
    SM = [sma, smb, smc]
    SS = [ssa, ssb, ssc]
    pltpu.sync_copy(row_hbm.at[pl.ds(base, EPW)], row_v)

    def zero_row(i, _):
        for j in range(F // 16):
            bufa[i, pl.ds(j * 16, 16)] = jnp.zeros((16,), jnp.float32)
        return ()

    lax.fori_loop(0, C, zero_row, ())

    def zfire(q, _):
        pltpu.async_copy(bufa, acc.at[pl.ds(s * RPT + q * C, C)], sga)
        return ()

    def zdrain(q, _):
        pltpu.make_async_copy(
            bufa, acc.at[pl.ds(s * RPT + q * C, C)], sga).wait()
        return ()

    @pl.when(s < 15)
    def _():
        lax.fori_loop(0, QN, zfire, ())
        lax.fori_loop(0, QN, zdrain, ())

    @pl.when(s == 15)
    def _():
        lax.fori_loop(0, QN_LAST, zfire, ())
        lax.fori_loop(0, QN_LAST, zdrain, ())

    plsc.subcore_barrier()

    def gstart(i, b):
        pltpu.async_copy(g_hbm.at[row_v.at[pl.ds(i * C, C)]], BUF[b], SG[b])
        pltpu.async_copy(col_hbm.at[pl.ds(base + i * C, C)], col_v.at[b], SM[b])
        pltpu.async_copy(ew_hbm.at[pl.ds(base + i * C, C)], w_v.at[b], SM[b])

    def sub(i, b):
        buf = BUF[b]
        pltpu.make_async_copy(
            g_hbm.at[row_v.at[pl.ds(i * C, C)]], buf, SG[b]).wait()
        pltpu.make_async_copy(
            col_hbm.at[pl.ds(base + i * C, C)], col_v.at[b], SM[b]).wait()
        pltpu.make_async_copy(
            ew_hbm.at[pl.ds(base + i * C, C)], w_v.at[b], SM[b]).wait()
        @plsc.parallel_loop(0, C // 16)
        def _scale(bi):
            wvec = w_v[b, pl.ds(bi * 16, 16)]
            for e16 in range(16):
                splat = _vsplat(wvec, e16)
                r = bi * 16 + e16
                for j in range(F // 16):
                    v = buf[r, pl.ds(j * 16, 16)]
                    buf[r, pl.ds(j * 16, 16)] = v * splat
        pltpu.async_copy(buf, acc.at[col_v.at[b]], SS[b], add=True)
        bn = (b + 2) % 3

        @pl.when((i >= 1) & (i + 2 < NCHUNK))
        def _():
            pltpu.make_async_copy(BUF[bn], acc.at[col_v.at[bn]], SS[bn]).wait()

        @pl.when(i + 2 < NCHUNK)
        def _():
            gstart(i + 2, bn)

    gstart(0, 0)
    gstart(1, 1)

    def tri(j, _):
        i0 = 3 * j
        for b in range(3):
            @pl.when(i0 + b < NCHUNK)
            def _(b=b):
                sub(i0 + b, b)
        return ()

    lax.fori_loop(0, (NCHUNK + 2) // 3, tri, ())
    for b in range(3):
        pltpu.make_async_copy(BUF[b], acc.at[col_v.at[b]], SS[b]).wait()
    plsc.subcore_barrier()

    def rb(nq):
        def stage(q):
            pltpu.async_copy(acc.at[pl.ds(s * RPT + q * C, C)], BUF[q % 3], sga)

        def flush(q):
            pltpu.make_async_copy(
                acc.at[pl.ds(s * RPT + q * C, C)], BUF[q % 3], sga).wait()
            pltpu.async_copy(
                BUF[q % 3], out_hbm.at[c, pl.ds(s * RPT + q * C, C)], SS[q % 3])

        def hwait(q):
            pltpu.make_async_copy(
                BUF[q % 3], out_hbm.at[c, pl.ds(s * RPT + q * C, C)],
                SS[q % 3]).wait()

        for q in range(nq):
            if q >= 3:
                hwait(q - 3)
            stage(q)
            flush(q)
        for q in range(max(0, nq - 3), nq):
            hwait(q)

    @pl.when(s < 15)
    def _():
        rb(QN)

    @pl.when(s == 15)
    def _():
        rb(QN_LAST)


_RB = 1000


def _dinv_body(d0_ref, d1_ref, out_ref, outsq_ref):
    d = d0_ref[...] + d1_ref[...]
    safe = jnp.where(d > 0, d, 1.0)
    di = jnp.where(d > 0, 1.0 / jnp.sqrt(safe), 0.0)
    out_ref[...] = di
    outsq_ref[...] = di * di


def _dinv(deg0, deg1):
    return pl.pallas_call(
        _dinv_body,
        out_shape=[jax.ShapeDtypeStruct((N,), jnp.float32),
                   jax.ShapeDtypeStruct((N,), jnp.float32)],
    )(deg0, deg1)


def _mm(h, w):
    return jnp.dot(h, w, preferred_element_type=jnp.float32)


def _tca_body(x_ref, di_ref, w_ref, out_ref, g_ref):
    x = x_ref[...]
    out_ref[...] = _mm(x, w_ref[...])
    g_ref[...] = di_ref[...] * x


def _tca(x, dinv2, w0):
    return pl.pallas_call(
        _tca_body,
        grid=(N // _RB,),
        in_specs=[
            pl.BlockSpec((_RB, F), lambda i: (i, 0)),
            pl.BlockSpec((_RB, 1), lambda i: (i, 0)),
            pl.BlockSpec((F, F), lambda i: (0, 0)),
        ],
        out_specs=[
            pl.BlockSpec((_RB, F), lambda i: (i, 0)),
            pl.BlockSpec((_RB, F), lambda i: (i, 0)),
        ],
        out_shape=[
            jax.ShapeDtypeStruct((N, F), jnp.float32),
            jax.ShapeDtypeStruct((N, F), jnp.float32),
        ],
    )(x, dinv2, w0)


def _tcg_body(p_ref, disq_ref, g_ref):
    g_ref[...] = disq_ref[...] * (p_ref[0] + p_ref[1])


def _tcg(p, dinvsq2):
    return pl.pallas_call(
        _tcg_body,
        grid=(N // _RB,),
        in_specs=[
            pl.BlockSpec((2, _RB, F), lambda i: (0, i, 0)),
            pl.BlockSpec((_RB, 1), lambda i: (i, 0)),
        ],
        out_specs=pl.BlockSpec((_RB, F), lambda i: (i, 0)),
        out_shape=jax.ShapeDtypeStruct((N, F), jnp.float32),
    )(p, dinvsq2)


def _tch_body(p_ref, di_ref, oprev_ref, w_ref, out_ref):
    hk = di_ref[...] * (p_ref[0] + p_ref[1])
    out_ref[...] = oprev_ref[...] + _mm(hk, w_ref[...])


def _tch(p, dinv2, out_prev, wk):
    return pl.pallas_call(
        _tch_body,
        grid=(N // _RB,),
        in_specs=[
            pl.BlockSpec((2, _RB, F), lambda i: (0, i, 0)),
            pl.BlockSpec((_RB, 1), lambda i: (i, 0)),
            pl.BlockSpec((_RB, F), lambda i: (i, 0)),
            pl.BlockSpec((F, F), lambda i: (0, 0)),
        ],
        out_specs=pl.BlockSpec((_RB, F), lambda i: (i, 0)),
        out_shape=jax.ShapeDtypeStruct((N, F), jnp.float32),
    )(p, dinv2, out_prev, wk)


def _tcc_body(act, p_ref, di_ref, oprev_ref, w_ref, b_ref, wn_ref,
              out_ref, g_ref):
    di = di_ref[...]
    hk = di * (p_ref[0] + p_ref[1])
    hnext = oprev_ref[...] + _mm(hk, w_ref[...]) + b_ref[...]
    if act:
        hnext = jnp.where(hnext > 0, hnext, 0.01 * hnext)
    out_ref[...] = _mm(hnext, wn_ref[...])
    g_ref[...] = di * hnext


def _tcc(p, dinv2, out_prev, w3, b2d, wn0, act):
    return pl.pallas_call(
        functools.partial(_tcc_body, act),
        grid=(N // _RB,),
        in_specs=[
            pl.BlockSpec((2, _RB, F), lambda i: (0, i, 0)),
            pl.BlockSpec((_RB, 1), lambda i: (i, 0)),
            pl.BlockSpec((_RB, F), lambda i: (i, 0)),
            pl.BlockSpec((F, F), lambda i: (0, 0)),
            pl.BlockSpec((1, F), lambda i: (0, 0)),
            pl.BlockSpec((F, F), lambda i: (0, 0)),
        ],
        out_specs=[
            pl.BlockSpec((_RB, F), lambda i: (i, 0)),
            pl.BlockSpec((_RB, F), lambda i: (i, 0)),
        ],
        out_shape=[
            jax.ShapeDtypeStruct((N, F), jnp.float32),
            jax.ShapeDtypeStruct((N, F), jnp.float32),
        ],
    )(p, dinv2, out_prev, w3, b2d, wn0)


def _tcd_body(p_ref, di_ref, oprev_ref, w_ref, b_ref, out_ref):
    hk = di_ref[...] * (p_ref[0] + p_ref[1])
    out_ref[...] = oprev_ref[...] + _mm(hk, w_ref[...]) + b_ref[...]


def _tcd(p, dinv2, out_prev, w3, b2d):
    return pl.pallas_call(
        _tcd_body,
        grid=(N // _RB,),
        in_specs=[
            pl.BlockSpec((2, _RB, F), lambda i: (0, i, 0)),
            pl.BlockSpec((_RB, 1), lambda i: (i, 0)),
            pl.BlockSpec((_RB, F), lambda i: (i, 0)),
            pl.BlockSpec((F, F), lambda i: (0, 0)),
            pl.BlockSpec((1, F), lambda i: (0, 0)),
        ],
        out_specs=pl.BlockSpec((_RB, F), lambda i: (i, 0)),
        out_shape=jax.ShapeDtypeStruct((N, F), jnp.float32),
    )(p, dinv2, out_prev, w3, b2d)


def kernel(x, edge_index, edge_weight, W, b):
    row = edge_index[0]
    col = edge_index[1]

    deg0, deg1 = _deg(col.reshape(NW, NCHUNK, C), edge_weight)
    dinv, dinvsq = _dinv(deg0, deg1)
    dinv2 = dinv[:, None]
    dinvsq2 = dinvsq[:, None]

    out, g = _tca(x, dinv2, W[0, 0])
    for l in range(L):
        for k in range(1, K + 1):
            p = _prop(g, row, col, edge_weight)
            if k < K:
                g = _tcg(p, dinvsq2)
                out = _tch(p, dinv2, out, W[l, k])
            elif l < L - 1:
                out, g = _tcc(p, dinv2, out, W[l, k], b[l][None, :],
                              W[l + 1, 0], act=(l == 0))
            else:
                return _tcd(p, dinv2, out, W[l, k], b[l][None, :])


# speedup vs baseline: 1.1909x; 1.1909x over previous
import functools

import jax
import jax.numpy as jnp
from jax import lax
from jax.experimental import pallas as pl
from jax.experimental.pallas import tpu as pltpu
from jax.experimental.pallas import tpu_sc as plsc

N = 10000
E = 320000
F = 128
K = 3
L = 3

NC = 2
NS = 16
NW = NC * NS
EPW = E // NW
C = 80
NCHUNK = EPW // C

RPT = 640
QN = RPT // C
QN_LAST = (N - 15 * RPT) // C

_MESH = plsc.VectorSubcoreMesh(
    core_axis_name="c", subcore_axis_name="s", num_cores=NC, num_subcores=NS)

_GD = lax.GatherDimensionNumbers(
    offset_dims=(), collapsed_slice_dims=(0,), start_index_map=(0,))


def _vsplat(vec, lane):
    idx = jnp.full((16,), lane, jnp.int32)
    return lax.gather(vec, idx[:, None], _GD, (1,),
                      mode=lax.GatherScatterMode.PROMISE_IN_BOUNDS)


def _staged(s, fn):
    @pl.when(s < 15)
    def _():
        lax.fori_loop(0, QN, lambda q, _: (fn(q),)[1:], ())

    @pl.when(s == 15)
    def _():
        lax.fori_loop(0, QN_LAST, lambda q, _: (fn(q),)[1:], ())


@functools.partial(
    pl.kernel,
    out_type=[jax.ShapeDtypeStruct((N,), jnp.float32),
              jax.ShapeDtypeStruct((N,), jnp.float32)],
    mesh=_MESH,
    scratch_types=[
        pltpu.VMEM((NCHUNK, C), jnp.int32),
        pltpu.VMEM((EPW,), jnp.float32),
        pltpu.VMEM((C,), jnp.float32),
        pltpu.SemaphoreType.DMA,
        pltpu.VMEM_SHARED((N,), jnp.float32),
    ],
)
def _deg(col3_hbm, ew_hbm, out0_hbm, out1_hbm, col_v, w_v, zb_v, sem, acc):
    c = lax.axis_index("c")
    s = lax.axis_index("s")
    wid = s * NC + c
    pltpu.sync_copy(col3_hbm.at[wid], col_v)
    pltpu.sync_copy(ew_hbm.at[pl.ds(wid * EPW, EPW)], w_v)
    for j in range(C // 16):
        zb_v[pl.ds(j * 16, 16)] = jnp.zeros((16,), jnp.float32)
    _staged(s, lambda q: pltpu.sync_copy(zb_v, acc.at[pl.ds(s * RPT + q * C, C)]))
    plsc.subcore_barrier()

    def fire(i, _):
        pltpu.async_copy(w_v.at[pl.ds(i * C, C)], acc.at[col_v.at[i]], sem,
                         add=True)
        return ()

    def drain(i, _):
        pltpu.make_async_copy(
            w_v.at[pl.ds(i * C, C)], acc.at[col_v.at[i]], sem).wait()
        return ()

    lax.fori_loop(0, NCHUNK, fire, ())
    lax.fori_loop(0, NCHUNK, drain, ())
    plsc.subcore_barrier()

    def rd(q):
        off = s * RPT + q * C
        pltpu.sync_copy(acc.at[pl.ds(off, C)], zb_v)

        @pl.when(c == 0)
        def _():
            pltpu.sync_copy(zb_v, out0_hbm.at[pl.ds(off, C)])

        @pl.when(c == 1)
        def _():
            pltpu.sync_copy(zb_v, out1_hbm.at[pl.ds(off, C)])

    _staged(s, rd)


@functools.partial(
    pl.kernel,
    out_type=jax.ShapeDtypeStruct((NC, N, F), jnp.float32),
    mesh=_MESH,
    scratch_types=[
        pltpu.VMEM((EPW,), jnp.int32),
        pltpu.VMEM((3, C), jnp.int32),
        pltpu.VMEM((3, C), jnp.float32),
        pltpu.VMEM((C, F), jnp.float32),
        pltpu.VMEM((C, F), jnp.float32),
        pltpu.VMEM((C, F), jnp.float32),
        pltpu.SemaphoreType.DMA,
        pltpu.SemaphoreType.DMA,
        pltpu.SemaphoreType.DMA,
        pltpu.SemaphoreType.DMA,
        pltpu.SemaphoreType.DMA,
        pltpu.SemaphoreType.DMA,
        pltpu.SemaphoreType.DMA,
        pltpu.SemaphoreType.DMA,
        pltpu.SemaphoreType.DMA,
        pltpu.VMEM_SHARED((N, F), jnp.float32),
    ],
)
def _prop(g_hbm, row_hbm, col_hbm, ew_hbm, out_hbm,
          row_v, col_v, w_v, bufa, bufb, bufc,
          sga, sgb, sgc, sma, smb, smc, ssa, ssb, ssc, acc):
    c = lax.axis_index("c")
    s = lax.axis_index("s")
    wid = s * NC + c
    base = wid * EPW
    BUF = [bufa, bufb, bufc]
    SG = [sga, sgb, sgc]
    SM = [sma, smb, smc]
    SS = [ssa, ssb, ssc]
    pltpu.sync_copy(row_hbm.at[pl.ds(base, EPW)], row_v)

    def zero_row(i, _):
        for j in range(F // 16):
            bufa[i, pl.ds(j * 16, 16)] = jnp.zeros((16,), jnp.float32)
        return ()

    lax.fori_loop(0, C, zero_row, ())

    def zfire(q, _):
        pltpu.async_copy(bufa, acc.at[pl.ds(s * RPT + q * C, C)], sga)
        return ()

    def zdrain(q, _):
        pltpu.make_async_copy(
            bufa, acc.at[pl.ds(s * RPT + q * C, C)], sga).wait()
        return ()

    @pl.when(s < 15)
    def _():
        lax.fori_loop(0, QN, zfire, ())
        lax.fori_loop(0, QN, zdrain, ())

    @pl.when(s == 15)
    def _():
        lax.fori_loop(0, QN_LAST, zfire, ())
        lax.fori_loop(0, QN_LAST, zdrain, ())

    plsc.subcore_barrier()

    def gstart(i, b):
        pltpu.async_copy(g_hbm.at[row_v.at[pl.ds(i * C, C)]], BUF[b], SG[b])
        pltpu.async_copy(col_hbm.at[pl.ds(base + i * C, C)], col_v.at[b], SM[b])
        pltpu.async_copy(ew_hbm.at[pl.ds(base + i * C, C)], w_v.at[b], SM[b])

    def sub(i, b):
        buf = BUF[b]
        pltpu.make_async_copy(
            g_hbm.at[row_v.at[pl.ds(i * C, C)]], buf, SG[b]).wait()
        pltpu.make_async_copy(
            col_hbm.at[pl.ds(base + i * C, C)], col_v.at[b], SM[b]).wait()
        pltpu.make_async_copy(
            ew_hbm.at[pl.ds(base + i * C, C)], w_v.at[b], SM[b]).wait()
        def scale_grp(bi, _):
            wvec = w_v[b, pl.ds(bi * 16, 16)]
            for e16 in range(16):
                splat = _vsplat(wvec, e16)
                r = bi * 16 + e16
                for j in range(F // 16):
                    v = buf[r, pl.ds(j * 16, 16)]
                    buf[r, pl.ds(j * 16, 16)] = v * splat
            return ()

        lax.fori_loop(0, C // 16, scale_grp, ())
        pltpu.async_copy(buf, acc.at[col_v.at[b]], SS[b], add=True)
        bn = (b + 2) % 3

        @pl.when((i >= 1) & (i + 2 < NCHUNK))
        def _():
            pltpu.make_async_copy(BUF[bn], acc.at[col_v.at[bn]], SS[bn]).wait()

        @pl.when(i + 2 < NCHUNK)
        def _():
            gstart(i + 2, bn)

    gstart(0, 0)
    gstart(1, 1)

    def tri(j, _):
        i0 = 3 * j
        for b in range(3):
            @pl.when(i0 + b < NCHUNK)
            def _(b=b):
                sub(i0 + b, b)
        return ()

    lax.fori_loop(0, (NCHUNK + 2) // 3, tri, ())
    for b in range(3):
        pltpu.make_async_copy(BUF[b], acc.at[col_v.at[b]], SS[b]).wait()
    plsc.subcore_barrier()

    def rb(nq):
        def stage(q):
            pltpu.async_copy(acc.at[pl.ds(s * RPT + q * C, C)], BUF[q % 3], sga)

        def flush(q):
            pltpu.make_async_copy(
                acc.at[pl.ds(s * RPT + q * C, C)], BUF[q % 3], sga).wait()
            pltpu.async_copy(
                BUF[q % 3], out_hbm.at[c, pl.ds(s * RPT + q * C, C)], SS[q % 3])

        def hwait(q):
            pltpu.make_async_copy(
                BUF[q % 3], out_hbm.at[c, pl.ds(s * RPT + q * C, C)],
                SS[q % 3]).wait()

        for q in range(nq):
            if q >= 3:
                hwait(q - 3)
            stage(q)
            flush(q)
        for q in range(max(0, nq - 3), nq):
            hwait(q)

    @pl.when(s < 15)
    def _():
        rb(QN)

    @pl.when(s == 15)
    def _():
        rb(QN_LAST)


_RB = 1000


def _dinv_body(d0_ref, d1_ref, out_ref, outsq_ref):
    d = d0_ref[...] + d1_ref[...]
    safe = jnp.where(d > 0, d, 1.0)
    di = jnp.where(d > 0, 1.0 / jnp.sqrt(safe), 0.0)
    out_ref[...] = di
    outsq_ref[...] = di * di


def _dinv(deg0, deg1):
    return pl.pallas_call(
        _dinv_body,
        out_shape=[jax.ShapeDtypeStruct((N,), jnp.float32),
                   jax.ShapeDtypeStruct((N,), jnp.float32)],
    )(deg0, deg1)


def _mm(h, w):
    return jnp.dot(h, w, preferred_element_type=jnp.float32)


def _tca_body(x_ref, di_ref, w_ref, out_ref, g_ref):
    x = x_ref[...]
    out_ref[...] = _mm(x, w_ref[...])
    g_ref[...] = di_ref[...] * x


def _tca(x, dinv2, w0):
    return pl.pallas_call(
        _tca_body,
        grid=(N // _RB,),
        in_specs=[
            pl.BlockSpec((_RB, F), lambda i: (i, 0)),
            pl.BlockSpec((_RB, 1), lambda i: (i, 0)),
            pl.BlockSpec((F, F), lambda i: (0, 0)),
        ],
        out_specs=[
            pl.BlockSpec((_RB, F), lambda i: (i, 0)),
            pl.BlockSpec((_RB, F), lambda i: (i, 0)),
        ],
        out_shape=[
            jax.ShapeDtypeStruct((N, F), jnp.float32),
            jax.ShapeDtypeStruct((N, F), jnp.float32),
        ],
    )(x, dinv2, w0)


def _tcg_body(p_ref, disq_ref, g_ref):
    g_ref[...] = disq_ref[...] * (p_ref[0] + p_ref[1])


def _tcg(p, dinvsq2):
    return pl.pallas_call(
        _tcg_body,
        grid=(N // _RB,),
        in_specs=[
            pl.BlockSpec((2, _RB, F), lambda i: (0, i, 0)),
            pl.BlockSpec((_RB, 1), lambda i: (i, 0)),
        ],
        out_specs=pl.BlockSpec((_RB, F), lambda i: (i, 0)),
        out_shape=jax.ShapeDtypeStruct((N, F), jnp.float32),
    )(p, dinvsq2)


def _tch_body(p_ref, di_ref, oprev_ref, w_ref, out_ref):
    hk = di_ref[...] * (p_ref[0] + p_ref[1])
    out_ref[...] = oprev_ref[...] + _mm(hk, w_ref[...])


def _tch(p, dinv2, out_prev, wk):
    return pl.pallas_call(
        _tch_body,
        grid=(N // _RB,),
        in_specs=[
            pl.BlockSpec((2, _RB, F), lambda i: (0, i, 0)),
            pl.BlockSpec((_RB, 1), lambda i: (i, 0)),
            pl.BlockSpec((_RB, F), lambda i: (i, 0)),
            pl.BlockSpec((F, F), lambda i: (0, 0)),
        ],
        out_specs=pl.BlockSpec((_RB, F), lambda i: (i, 0)),
        out_shape=jax.ShapeDtypeStruct((N, F), jnp.float32),
    )(p, dinv2, out_prev, wk)


def _tcc_body(act, p_ref, di_ref, oprev_ref, w_ref, b_ref, wn_ref,
              out_ref, g_ref):
    di = di_ref[...]
    hk = di * (p_ref[0] + p_ref[1])
    hnext = oprev_ref[...] + _mm(hk, w_ref[...]) + b_ref[...]
    if act:
        hnext = jnp.where(hnext > 0, hnext, 0.01 * hnext)
    out_ref[...] = _mm(hnext, wn_ref[...])
    g_ref[...] = di * hnext


def _tcc(p, dinv2, out_prev, w3, b2d, wn0, act):
    return pl.pallas_call(
        functools.partial(_tcc_body, act),
        grid=(N // _RB,),
        in_specs=[
            pl.BlockSpec((2, _RB, F), lambda i: (0, i, 0)),
            pl.BlockSpec((_RB, 1), lambda i: (i, 0)),
            pl.BlockSpec((_RB, F), lambda i: (i, 0)),
            pl.BlockSpec((F, F), lambda i: (0, 0)),
            pl.BlockSpec((1, F), lambda i: (0, 0)),
            pl.BlockSpec((F, F), lambda i: (0, 0)),
        ],
        out_specs=[
            pl.BlockSpec((_RB, F), lambda i: (i, 0)),
            pl.BlockSpec((_RB, F), lambda i: (i, 0)),
        ],
        out_shape=[
            jax.ShapeDtypeStruct((N, F), jnp.float32),
            jax.ShapeDtypeStruct((N, F), jnp.float32),
        ],
    )(p, dinv2, out_prev, w3, b2d, wn0)


def _tcd_body(p_ref, di_ref, oprev_ref, w_ref, b_ref, out_ref):
    hk = di_ref[...] * (p_ref[0] + p_ref[1])
    out_ref[...] = oprev_ref[...] + _mm(hk, w_ref[...]) + b_ref[...]


def _tcd(p, dinv2, out_prev, w3, b2d):
    return pl.pallas_call(
        _tcd_body,
        grid=(N // _RB,),
        in_specs=[
            pl.BlockSpec((2, _RB, F), lambda i: (0, i, 0)),
            pl.BlockSpec((_RB, 1), lambda i: (i, 0)),
            pl.BlockSpec((_RB, F), lambda i: (i, 0)),
            pl.BlockSpec((F, F), lambda i: (0, 0)),
            pl.BlockSpec((1, F), lambda i: (0, 0)),
        ],
        out_specs=pl.BlockSpec((_RB, F), lambda i: (i, 0)),
        out_shape=jax.ShapeDtypeStruct((N, F), jnp.float32),
    )(p, dinv2, out_prev, w3, b2d)


def kernel(x, edge_index, edge_weight, W, b):
    row = edge_index[0]
    col = edge_index[1]

    deg0, deg1 = _deg(col.reshape(NW, NCHUNK, C), edge_weight)
    dinv, dinvsq = _dinv(deg0, deg1)
    dinv2 = dinv[:, None]
    dinvsq2 = dinvsq[:, None]

    out, g = _tca(x, dinv2, W[0, 0])
    for l in range(L):
        for k in range(1, K + 1):
            p = _prop(g, row, col, edge_weight)
            if k < K:
                g = _tcg(p, dinvsq2)
                out = _tch(p, dinv2, out, W[l, k])
            elif l < L - 1:
                out, g = _tcc(p, dinv2, out, W[l, k], b[l][None, :],
                              W[l + 1, 0], act=(l == 0))
            else:
                return _tcd(p, dinv2, out, W[l, k], b[l][None, :])
